# baseline (speedup 1.0000x reference)
import functools
import math

import numpy as np
import jax
import jax.numpy as jnp
from jax import lax
from jax.experimental import pallas as pl
from jax.experimental.pallas import tpu as pltpu
from jax.experimental.pallas import tpu_sc as plsc

N = 10000
E = 160000
D = 128
NB = 8
NH = 64

NCORE = 2
NSUB = 16
CH = 96
NCHUNK = 54
EPC = NCHUNK * CH
EPAD = NCORE * NSUB * EPC
NPAD = 10112
ROWS_PER_TILE = NPAD // NSUB
NROT = 3

_z = np.random.default_rng(0).standard_normal(1000000)
_ACT = float(1.0 / np.sqrt(np.mean((_z / (1.0 + np.exp(-_z))) ** 2)))
C_S = math.sin(math.pi / 8.0)
C_X = math.cos(math.pi / 8.0)


def _w_body(el_ref, ea_ref, w1_ref, w2_ref, o_ref):
    h = jnp.dot(el_ref[...], w1_ref[...], preferred_element_type=jnp.float32)
    h = h * (1.0 / math.sqrt(NB))
    h = h * jax.nn.sigmoid(h) * _ACT
    w = jnp.dot(h, w2_ref[...], preferred_element_type=jnp.float32)
    w = w * ea_ref[...] * (0.25 / math.sqrt(NH))
    bits = jax.lax.bitcast_convert_type(w, jnp.uint32)
    lo = (bits[:, :64] + jnp.uint32(0x8000)) >> jnp.uint32(16)
    hi = (bits[:, 64:] + jnp.uint32(0x8000)) & jnp.uint32(0xFFFF0000)
    o_ref[...] = (hi | lo).astype(jnp.int32)


def _edge_weights():
    BE = 2048
    return pl.pallas_call(
        _w_body,
        grid=(EPAD // BE,),
        in_specs=[
            pl.BlockSpec((BE, NB), lambda i: (i, 0)),
            pl.BlockSpec((BE, 1), lambda i: (i, 0)),
            pl.BlockSpec((NB, NH), lambda i: (0, 0)),
            pl.BlockSpec((NH, D), lambda i: (0, 0)),
        ],
        out_specs=pl.BlockSpec((BE, D // 2), lambda i: (i, 0)),
        out_shape=jax.ShapeDtypeStruct((EPAD, D // 2), jnp.int32),
    )


def _x_body(ni_ref, na_ref, w_ref, o_ref):
    o_ref[...] = jnp.dot(ni_ref[...] * na_ref[...], w_ref[...],
                         preferred_element_type=jnp.float32) * (1.0 / math.sqrt(D))


def _node_lin(ni, na, w):
    BN = 2000
    return pl.pallas_call(
        _x_body,
        grid=(N // BN,),
        in_specs=[
            pl.BlockSpec((BN, D), lambda i: (i, 0)),
            pl.BlockSpec((BN, 1), lambda i: (i, 0)),
            pl.BlockSpec((D, D), lambda i: (0, 0)),
        ],
        out_specs=pl.BlockSpec((BN, D), lambda i: (i, 0)),
        out_shape=jax.ShapeDtypeStruct((N, D), jnp.float32),
    )(ni, na, w)


_mesh = plsc.VectorSubcoreMesh(core_axis_name="c", subcore_axis_name="s")


@functools.partial(
    pl.kernel,
    out_type=jax.ShapeDtypeStruct((NCORE, NPAD, D), jnp.float32),
    mesh=_mesh,
    scratch_types=[
        pltpu.VMEM_SHARED((NPAD, D), jnp.float32),
        pltpu.VMEM((NROT, CH), jnp.int32),
        pltpu.VMEM((NROT, CH), jnp.int32),
        pltpu.VMEM((CH, D // 2), jnp.int32),
        pltpu.VMEM((NROT, CH, D), jnp.float32),
        pltpu.SemaphoreType.DMA,
        pltpu.SemaphoreType.DMA,
        pltpu.SemaphoreType.DMA,
        pltpu.SemaphoreType.DMA,
        pltpu.SemaphoreType.DMA,
    ],
)
def _sc_scatter(x_hbm, src_hbm, dst_hbm, w_hbm, out_hbm,
                agg, src_r, dst_r, wbuf, gbuf, isem, dsem, wsem, gsem, ssem):
    cid = lax.axis_index("c")
    sid = lax.axis_index("s")

    for b in range(NROT):
        pltpu.async_copy(src_hbm.at[cid, sid, b], src_r.at[b], isem).wait()
        pltpu.async_copy(dst_hbm.at[cid, sid, b], dst_r.at[b], dsem).wait()

    pltpu.async_copy(w_hbm.at[cid, sid, pl.ds(0, CH)], wbuf, wsem)
    pltpu.async_copy(x_hbm.at[src_r.at[0]], gbuf.at[0], gsem)
    pltpu.async_copy(x_hbm.at[src_r.at[1]], gbuf.at[1], gsem)

    zeros16 = jnp.zeros((16,), jnp.float32)

    @plsc.parallel_loop(0, CH)
    def _zb(r):
        for j in range(8):
            gbuf[NROT - 1, r, pl.ds(j * 16, 16)] = zeros16

    base = sid * ROWS_PER_TILE
    for k in range(ROWS_PER_TILE // CH):
        pltpu.sync_copy(gbuf.at[NROT - 1], agg.at[pl.ds(base + k * CH, CH)])
    rem = ROWS_PER_TILE % CH
    if rem:
        pltpu.sync_copy(gbuf.at[NROT - 1, pl.ds(0, rem)],
                        agg.at[pl.ds(base + ROWS_PER_TILE - rem, rem)])
    plsc.subcore_barrier()

    def _outer(i3, carry):
        for b in range(NROT):
            c = i3 * NROT + b
            b2 = (b + 2) % NROT
            pltpu.make_async_copy(
                w_hbm.at[cid, sid, pl.ds(c * CH, CH)], wbuf, wsem).wait()
            pltpu.make_async_copy(
                x_hbm.at[src_r.at[b]], gbuf.at[b], gsem).wait()

            @pl.when(c + NROT < NCHUNK)
            def _():
                pltpu.async_copy(
                    src_hbm.at[cid, sid, c + NROT], src_r.at[b], isem)

            shift16 = jnp.full((16,), 16, jnp.int32)
            mask_hi = jnp.full((16,), -65536, jnp.int32)

            @plsc.parallel_loop(0, CH)
            def _mb(r):
                for j in range(4):
                    wv = wbuf[r, pl.ds(j * 16, 16)]
                    wlo = lax.bitcast_convert_type(
                        lax.shift_left(wv, shift16), jnp.float32)
                    whi = lax.bitcast_convert_type(
                        lax.bitwise_and(wv, mask_hi), jnp.float32)
                    slo = pl.ds(j * 16, 16)
                    shi = pl.ds(64 + j * 16, 16)
                    gbuf[b, r, slo] = gbuf[b, r, slo] * wlo
                    gbuf[b, r, shi] = gbuf[b, r, shi] * whi

            @pl.when(c + 1 < NCHUNK)
            def _():
                pltpu.async_copy(
                    w_hbm.at[cid, sid, pl.ds((c + 1) * CH, CH)], wbuf, wsem)

            @pl.when(c >= 1)
            def _():
                pltpu.make_async_copy(
                    gbuf.at[b2], agg.at[dst_r.at[b2]], ssem).wait()

                @pl.when(c + 2 < NCHUNK)
                def _():
                    pltpu.async_copy(
                        dst_hbm.at[cid, sid, c + 2], dst_r.at[b2], dsem)

            @pl.when(c + 2 < NCHUNK)
            def _():
                @pl.when(c >= 1)
                def _():
                    pltpu.make_async_copy(
                        src_hbm.at[cid, sid, c + 2], src_r.at[b2], isem).wait()

                pltpu.async_copy(x_hbm.at[src_r.at[b2]], gbuf.at[b2], gsem)

            @pl.when(c >= NROT)
            def _():
                pltpu.make_async_copy(
                    dst_hbm.at[cid, sid, c], dst_r.at[b], dsem).wait()

            pltpu.async_copy(gbuf.at[b], agg.at[dst_r.at[b]], ssem, add=True)
        return carry

    lax.fori_loop(0, NCHUNK // NROT, _outer, 0)

    pltpu.make_async_copy(
        gbuf.at[(NCHUNK - 1) % NROT],
        agg.at[dst_r.at[(NCHUNK - 1) % NROT]], ssem).wait()
    plsc.subcore_barrier()

    pltpu.sync_copy(agg.at[pl.ds(base, ROWS_PER_TILE)],
                    out_hbm.at[cid, pl.ds(base, ROWS_PER_TILE)])


def _f_body(ni_ref, na_ref, agg_ref, wsc_ref, wl2_ref, o_ref):
    na = na_ref[...]
    nie = ni_ref[...] * na
    aggs = (agg_ref[0] + agg_ref[1]) * na
    o_ref[...] = (jnp.dot(nie, wsc_ref[...], preferred_element_type=jnp.float32)
                  * (C_S / math.sqrt(D))
                  + jnp.dot(aggs, wl2_ref[...], preferred_element_type=jnp.float32)
                  * (C_X / math.sqrt(D)))


def _final(ni, na, agg2, w_sc, w_lin2):
    BN = 2000
    return pl.pallas_call(
        _f_body,
        grid=(N // BN,),
        in_specs=[
            pl.BlockSpec((BN, D), lambda i: (i, 0)),
            pl.BlockSpec((BN, 1), lambda i: (i, 0)),
            pl.BlockSpec((NCORE, BN, D), lambda i: (0, i, 0)),
            pl.BlockSpec((D, D), lambda i: (0, 0)),
            pl.BlockSpec((D, D), lambda i: (0, 0)),
        ],
        out_specs=pl.BlockSpec((BN, D), lambda i: (i, 0)),
        out_shape=jax.ShapeDtypeStruct((N, D), jnp.float32),
    )(ni, na, agg2, w_sc, w_lin2)


def kernel(node_input, node_attr, edge_src, edge_dst, edge_attr,
           edge_length_embedded, w_sc, w_lin1, w_lin2, fc_w1, fc_w2):
    pad = EPAD - E
    src4 = jnp.reshape(
        jnp.concatenate([edge_src, jnp.zeros((pad,), jnp.int32)]),
        (NCORE, NSUB, NCHUNK, CH))
    dst4 = jnp.reshape(
        jnp.concatenate([edge_dst, jnp.zeros((pad,), jnp.int32)]),
        (NCORE, NSUB, NCHUNK, CH))
    ea_p = jnp.concatenate([edge_attr, jnp.zeros((pad, 1), jnp.float32)])
    el_p = jnp.concatenate(
        [edge_length_embedded, jnp.zeros((pad, NB), jnp.float32)])

    w_edges = _edge_weights()(el_p, ea_p, fc_w1, fc_w2)
    w4 = jnp.reshape(w_edges, (NCORE, NSUB, EPC, D // 2))
    x = _node_lin(node_input, node_attr, w_lin1)
    agg2 = _sc_scatter(x, src4, dst4, w4)
    return _final(node_input, node_attr, agg2[:, :N], w_sc, w_lin2)


# speedup vs baseline: 1.1556x; 1.1556x over previous
import functools
import math

import numpy as np
import jax
import jax.numpy as jnp
from jax import lax
from jax.experimental import pallas as pl
from jax.experimental.pallas import tpu as pltpu
from jax.experimental.pallas import tpu_sc as plsc

N = 10000
E = 160000
D = 128
NB = 8
NH = 64

NCORE = 2
NSUB = 16
CH = 64
NCHUNK = 80
EPC = NCHUNK * CH
EPAD = NCORE * NSUB * EPC
NPAD = 10112
ROWS_PER_TILE = NPAD // NSUB
NROT = 4

_z = np.random.default_rng(0).standard_normal(1000000)
_ACT = float(1.0 / np.sqrt(np.mean((_z / (1.0 + np.exp(-_z))) ** 2)))
C_S = math.sin(math.pi / 8.0)
C_X = math.cos(math.pi / 8.0)


def _w_body(el_ref, ea_ref, w1_ref, w2_ref, o_ref):
    h = jnp.dot(el_ref[...], w1_ref[...], preferred_element_type=jnp.float32)
    h = h * (1.0 / math.sqrt(NB))
    h = h * jax.nn.sigmoid(h) * _ACT
    w = jnp.dot(h, w2_ref[...], preferred_element_type=jnp.float32)
    w = w * ea_ref[...] * (0.25 / math.sqrt(NH))
    bits = jax.lax.bitcast_convert_type(w, jnp.uint32)
    lo = (bits[:, :64] + jnp.uint32(0x8000)) >> jnp.uint32(16)
    hi = (bits[:, 64:] + jnp.uint32(0x8000)) & jnp.uint32(0xFFFF0000)
    o_ref[...] = (hi | lo).astype(jnp.int32)


def _edge_weights():
    BE = 2048
    return pl.pallas_call(
        _w_body,
        grid=(EPAD // BE,),
        in_specs=[
            pl.BlockSpec((BE, NB), lambda i: (i, 0)),
            pl.BlockSpec((BE, 1), lambda i: (i, 0)),
            pl.BlockSpec((NB, NH), lambda i: (0, 0)),
            pl.BlockSpec((NH, D), lambda i: (0, 0)),
        ],
        out_specs=pl.BlockSpec((BE, D // 2), lambda i: (i, 0)),
        out_shape=jax.ShapeDtypeStruct((EPAD, D // 2), jnp.int32),
    )


def _x_body(ni_ref, na_ref, w_ref, o_ref):
    o_ref[...] = jnp.dot(ni_ref[...] * na_ref[...], w_ref[...],
                         preferred_element_type=jnp.float32) * (1.0 / math.sqrt(D))


def _node_lin(ni, na, w):
    BN = 2000
    return pl.pallas_call(
        _x_body,
        grid=(N // BN,),
        in_specs=[
            pl.BlockSpec((BN, D), lambda i: (i, 0)),
            pl.BlockSpec((BN, 1), lambda i: (i, 0)),
            pl.BlockSpec((D, D), lambda i: (0, 0)),
        ],
        out_specs=pl.BlockSpec((BN, D), lambda i: (i, 0)),
        out_shape=jax.ShapeDtypeStruct((N, D), jnp.float32),
    )(ni, na, w)


_mesh = plsc.VectorSubcoreMesh(core_axis_name="c", subcore_axis_name="s")


@functools.partial(
    pl.kernel,
    out_type=jax.ShapeDtypeStruct((NCORE, NPAD, D), jnp.float32),
    mesh=_mesh,
    scratch_types=[
        pltpu.VMEM_SHARED((NPAD, D), jnp.float32),
        pltpu.VMEM((NROT, CH), jnp.int32),
        pltpu.VMEM((NROT, CH), jnp.int32),
        pltpu.VMEM((2, CH, D // 2), jnp.int32),
        pltpu.VMEM((NROT, CH, D), jnp.float32),
        pltpu.SemaphoreType.DMA,
        pltpu.SemaphoreType.DMA,
        pltpu.SemaphoreType.DMA,
        pltpu.SemaphoreType.DMA,
        pltpu.SemaphoreType.DMA,
    ],
)
def _sc_scatter(x_hbm, src_hbm, dst_hbm, w_hbm, out_hbm,
                agg, src_r, dst_r, wbuf, gbuf, isem, dsem, wsem, gsem, ssem):
    cid = lax.axis_index("c")
    sid = lax.axis_index("s")

    for p in range(2):
        pltpu.async_copy(src_hbm.at[cid, sid, pl.ds(2 * p, 2)],
                         src_r.at[pl.ds(2 * p, 2)], isem).wait()
        pltpu.async_copy(dst_hbm.at[cid, sid, pl.ds(2 * p, 2)],
                         dst_r.at[pl.ds(2 * p, 2)], dsem).wait()

    pltpu.async_copy(w_hbm.at[cid, sid, pl.ds(0, CH)], wbuf.at[0], wsem)
    pltpu.async_copy(w_hbm.at[cid, sid, pl.ds(CH, CH)], wbuf.at[1], wsem)
    pltpu.async_copy(x_hbm.at[src_r.at[0]], gbuf.at[0], gsem)
    pltpu.async_copy(x_hbm.at[src_r.at[1]], gbuf.at[1], gsem)

    zeros16 = jnp.zeros((16,), jnp.float32)

    @plsc.parallel_loop(0, CH)
    def _zb(r):
        for j in range(8):
            gbuf[NROT - 1, r, pl.ds(j * 16, 16)] = zeros16

    base = sid * ROWS_PER_TILE
    for k in range(ROWS_PER_TILE // CH):
        pltpu.sync_copy(gbuf.at[NROT - 1], agg.at[pl.ds(base + k * CH, CH)])
    _rem = ROWS_PER_TILE % CH
    if _rem:
        pltpu.sync_copy(gbuf.at[NROT - 1, pl.ds(0, _rem)],
                        agg.at[pl.ds(base + ROWS_PER_TILE - _rem, _rem)])
    plsc.subcore_barrier()

    def _outer(i4, carry):
        for b in range(NROT):
            c = i4 * NROT + b
            b2 = (b + 2) % NROT
            ws = b % 2
            pltpu.make_async_copy(
                w_hbm.at[cid, sid, pl.ds(c * CH, CH)], wbuf.at[ws], wsem).wait()
            pltpu.make_async_copy(
                x_hbm.at[src_r.at[b]], gbuf.at[b], gsem).wait()

            shift16 = jnp.full((16,), 16, jnp.int32)
            mask_hi = jnp.full((16,), -65536, jnp.int32)

            @plsc.parallel_loop(0, CH)
            def _mb(r):
                for j in range(4):
                    wv = wbuf[ws, r, pl.ds(j * 16, 16)]
                    wlo = lax.bitcast_convert_type(
                        lax.shift_left(wv, shift16), jnp.float32)
                    whi = lax.bitcast_convert_type(
                        lax.bitwise_and(wv, mask_hi), jnp.float32)
                    slo = pl.ds(j * 16, 16)
                    shi = pl.ds(64 + j * 16, 16)
                    gbuf[b, r, slo] = gbuf[b, r, slo] * wlo
                    gbuf[b, r, shi] = gbuf[b, r, shi] * whi

            @pl.when(c + 2 < NCHUNK)
            def _():
                pltpu.async_copy(
                    w_hbm.at[cid, sid, pl.ds((c + 2) * CH, CH)],
                    wbuf.at[ws], wsem)

            @pl.when(c >= 2)
            def _():
                pltpu.make_async_copy(
                    gbuf.at[b2], agg.at[dst_r.at[b2]], ssem).wait()

            if b % 2 == 1:
                @pl.when((c >= 3) & (c + 2 < NCHUNK))
                def _():
                    pltpu.async_copy(
                        dst_hbm.at[cid, sid, pl.ds(c + 1, 2)],
                        dst_r.at[pl.ds((b + 1) % NROT, 2)], dsem)

                @pl.when(c + 4 < NCHUNK)
                def _():
                    pltpu.async_copy(
                        src_hbm.at[cid, sid, pl.ds(c + 3, 2)],
                        src_r.at[pl.ds((b + 3) % NROT, 2)], isem)

            @pl.when(c + 2 < NCHUNK)
            def _():
                if b % 2 == 0:
                    @pl.when(c >= 2)
                    def _():
                        pltpu.make_async_copy(
                            src_hbm.at[cid, sid, pl.ds(c + 2, 2)],
                            src_r.at[pl.ds(b2, 2)], isem).wait()

                pltpu.async_copy(x_hbm.at[src_r.at[b2]], gbuf.at[b2], gsem)

            if b % 2 == 0:
                @pl.when(c >= 4)
                def _():
                    pltpu.make_async_copy(
                        dst_hbm.at[cid, sid, pl.ds(c, 2)],
                        dst_r.at[pl.ds(b, 2)], dsem).wait()

            pltpu.async_copy(gbuf.at[b], agg.at[dst_r.at[b]], ssem, add=True)
        return carry

    lax.fori_loop(0, NCHUNK // NROT, _outer, 0)

    pltpu.make_async_copy(
        gbuf.at[(NCHUNK - 2) % NROT],
        agg.at[dst_r.at[(NCHUNK - 2) % NROT]], ssem).wait()
    pltpu.make_async_copy(
        gbuf.at[(NCHUNK - 1) % NROT],
        agg.at[dst_r.at[(NCHUNK - 1) % NROT]], ssem).wait()
    plsc.subcore_barrier()

    pltpu.sync_copy(agg.at[pl.ds(base, ROWS_PER_TILE)],
                    out_hbm.at[cid, pl.ds(base, ROWS_PER_TILE)])


def _f_body(ni_ref, na_ref, agg_ref, wsc_ref, wl2_ref, o_ref):
    na = na_ref[...]
    nie = ni_ref[...] * na
    aggs = (agg_ref[0] + agg_ref[1]) * na
    o_ref[...] = (jnp.dot(nie, wsc_ref[...], preferred_element_type=jnp.float32)
                  * (C_S / math.sqrt(D))
                  + jnp.dot(aggs, wl2_ref[...], preferred_element_type=jnp.float32)
                  * (C_X / math.sqrt(D)))


def _final(ni, na, agg2, w_sc, w_lin2):
    BN = 2000
    return pl.pallas_call(
        _f_body,
        grid=(N // BN,),
        in_specs=[
            pl.BlockSpec((BN, D), lambda i: (i, 0)),
            pl.BlockSpec((BN, 1), lambda i: (i, 0)),
            pl.BlockSpec((NCORE, BN, D), lambda i: (0, i, 0)),
            pl.BlockSpec((D, D), lambda i: (0, 0)),
            pl.BlockSpec((D, D), lambda i: (0, 0)),
        ],
        out_specs=pl.BlockSpec((BN, D), lambda i: (i, 0)),
        out_shape=jax.ShapeDtypeStruct((N, D), jnp.float32),
    )(ni, na, agg2, w_sc, w_lin2)


def kernel(node_input, node_attr, edge_src, edge_dst, edge_attr,
           edge_length_embedded, w_sc, w_lin1, w_lin2, fc_w1, fc_w2):
    pad = EPAD - E
    src4 = jnp.reshape(
        jnp.concatenate([edge_src, jnp.zeros((pad,), jnp.int32)]),
        (NCORE, NSUB, NCHUNK, CH))
    dst4 = jnp.reshape(
        jnp.concatenate([edge_dst, jnp.zeros((pad,), jnp.int32)]),
        (NCORE, NSUB, NCHUNK, CH))
    ea_p = jnp.concatenate([edge_attr, jnp.zeros((pad, 1), jnp.float32)])
    el_p = jnp.concatenate(
        [edge_length_embedded, jnp.zeros((pad, NB), jnp.float32)])

    w_edges = _edge_weights()(el_p, ea_p, fc_w1, fc_w2)
    w4 = jnp.reshape(w_edges, (NCORE, NSUB, EPC, D // 2))
    x = _node_lin(node_input, node_attr, w_lin1)
    agg2 = _sc_scatter(x, src4, dst4, w4)
    return _final(node_input, node_attr, agg2[:, :N], w_sc, w_lin2)


# speedup vs baseline: 1.1738x; 1.0157x over previous
import functools
import math

import numpy as np
import jax
import jax.numpy as jnp
from jax import lax
from jax.experimental import pallas as pl
from jax.experimental.pallas import tpu as pltpu
from jax.experimental.pallas import tpu_sc as plsc

N = 10000
E = 160000
D = 128
NB = 8
NH = 64

NCORE = 2
NSUB = 16
CH = 64
NCHUNK = 80
EPC = NCHUNK * CH
EPAD = NCORE * NSUB * EPC
NPAD = 10112
ROWS_PER_TILE = NPAD // NSUB
NROT = 4

_z = np.random.default_rng(0).standard_normal(1000000)
_ACT = float(1.0 / np.sqrt(np.mean((_z / (1.0 + np.exp(-_z))) ** 2)))
C_S = math.sin(math.pi / 8.0)
C_X = math.cos(math.pi / 8.0)


def _w_body(el_ref, ea_ref, w1_ref, w2_ref, ni_ref, na_ref, wl1_ref,
            o_ref, x_ref):
    h = jnp.dot(el_ref[...], w1_ref[...], preferred_element_type=jnp.float32)
    h = h * (1.0 / math.sqrt(NB))
    h = h * jax.nn.sigmoid(h) * _ACT
    w = jnp.dot(h, w2_ref[...], preferred_element_type=jnp.float32)
    w = w * ea_ref[...] * (0.25 / math.sqrt(NH))
    bits = jax.lax.bitcast_convert_type(w, jnp.uint32)
    lo = (bits[:, :64] + jnp.uint32(0x8000)) >> jnp.uint32(16)
    hi = (bits[:, 64:] + jnp.uint32(0x8000)) & jnp.uint32(0xFFFF0000)
    o_ref[...] = (hi | lo).astype(jnp.int32)
    x_ref[...] = jnp.dot(ni_ref[...] * na_ref[...], wl1_ref[...],
                         preferred_element_type=jnp.float32) * (1.0 / math.sqrt(D))


_NXB = 25


def _edge_weights():
    BE = 2048
    nmap = lambda i: (jnp.minimum(i, _NXB - 1), 0)
    return pl.pallas_call(
        _w_body,
        grid=(EPAD // BE,),
        in_specs=[
            pl.BlockSpec((BE, NB), lambda i: (i, 0)),
            pl.BlockSpec((BE, 1), lambda i: (i, 0)),
            pl.BlockSpec((NB, NH), lambda i: (0, 0)),
            pl.BlockSpec((NH, D), lambda i: (0, 0)),
            pl.BlockSpec((N // _NXB, D), nmap),
            pl.BlockSpec((N // _NXB, 1), nmap),
            pl.BlockSpec((D, D), lambda i: (0, 0)),
        ],
        out_specs=[
            pl.BlockSpec((BE, D // 2), lambda i: (i, 0)),
            pl.BlockSpec((N // _NXB, D), nmap),
        ],
        out_shape=[
            jax.ShapeDtypeStruct((EPAD, D // 2), jnp.int32),
            jax.ShapeDtypeStruct((N, D), jnp.float32),
        ],
    )


def _x_body(ni_ref, na_ref, w_ref, o_ref):
    o_ref[...] = jnp.dot(ni_ref[...] * na_ref[...], w_ref[...],
                         preferred_element_type=jnp.float32) * (1.0 / math.sqrt(D))


def _node_lin(ni, na, w):
    BN = 2000
    return pl.pallas_call(
        _x_body,
        grid=(N // BN,),
        in_specs=[
            pl.BlockSpec((BN, D), lambda i: (i, 0)),
            pl.BlockSpec((BN, 1), lambda i: (i, 0)),
            pl.BlockSpec((D, D), lambda i: (0, 0)),
        ],
        out_specs=pl.BlockSpec((BN, D), lambda i: (i, 0)),
        out_shape=jax.ShapeDtypeStruct((N, D), jnp.float32),
    )(ni, na, w)


_mesh = plsc.VectorSubcoreMesh(core_axis_name="c", subcore_axis_name="s")


@functools.partial(
    pl.kernel,
    out_type=jax.ShapeDtypeStruct((NCORE, NPAD, D), jnp.float32),
    mesh=_mesh,
    scratch_types=[
        pltpu.VMEM_SHARED((NPAD, D), jnp.float32),
        pltpu.VMEM((NROT, CH), jnp.int32),
        pltpu.VMEM((NROT, CH), jnp.int32),
        pltpu.VMEM((2, CH, D // 2), jnp.int32),
        pltpu.VMEM((NROT, CH, D), jnp.float32),
        pltpu.SemaphoreType.DMA,
        pltpu.SemaphoreType.DMA,
        pltpu.SemaphoreType.DMA,
        pltpu.SemaphoreType.DMA,
        pltpu.SemaphoreType.DMA,
    ],
)
def _sc_scatter(x_hbm, src_hbm, dst_hbm, w_hbm, out_hbm,
                agg, src_r, dst_r, wbuf, gbuf, isem, dsem, wsem, gsem, ssem):
    cid = lax.axis_index("c")
    sid = lax.axis_index("s")

    for p in range(2):
        pltpu.async_copy(src_hbm.at[cid, sid, pl.ds(2 * p, 2)],
                         src_r.at[pl.ds(2 * p, 2)], isem).wait()
        pltpu.async_copy(dst_hbm.at[cid, sid, pl.ds(2 * p, 2)],
                         dst_r.at[pl.ds(2 * p, 2)], dsem).wait()

    pltpu.async_copy(w_hbm.at[cid, sid, pl.ds(0, CH)], wbuf.at[0], wsem)
    pltpu.async_copy(w_hbm.at[cid, sid, pl.ds(CH, CH)], wbuf.at[1], wsem)
    pltpu.async_copy(x_hbm.at[src_r.at[0]], gbuf.at[0], gsem)
    pltpu.async_copy(x_hbm.at[src_r.at[1]], gbuf.at[1], gsem)

    zeros16 = jnp.zeros((16,), jnp.float32)

    @plsc.parallel_loop(0, CH)
    def _zb(r):
        for j in range(8):
            gbuf[NROT - 1, r, pl.ds(j * 16, 16)] = zeros16

    base = sid * ROWS_PER_TILE
    for k in range(ROWS_PER_TILE // CH):
        pltpu.sync_copy(gbuf.at[NROT - 1], agg.at[pl.ds(base + k * CH, CH)])
    _rem = ROWS_PER_TILE % CH
    if _rem:
        pltpu.sync_copy(gbuf.at[NROT - 1, pl.ds(0, _rem)],
                        agg.at[pl.ds(base + ROWS_PER_TILE - _rem, _rem)])
    plsc.subcore_barrier()

    def _outer(i4, carry):
        for b in range(NROT):
            c = i4 * NROT + b
            b2 = (b + 2) % NROT
            ws = b % 2
            pltpu.make_async_copy(
                w_hbm.at[cid, sid, pl.ds(c * CH, CH)], wbuf.at[ws], wsem).wait()
            pltpu.make_async_copy(
                x_hbm.at[src_r.at[b]], gbuf.at[b], gsem).wait()

            shift16 = jnp.full((16,), 16, jnp.int32)
            mask_hi = jnp.full((16,), -65536, jnp.int32)

            @plsc.parallel_loop(0, CH)
            def _mb(r):
                for j in range(4):
                    wv = wbuf[ws, r, pl.ds(j * 16, 16)]
                    wlo = lax.bitcast_convert_type(
                        lax.shift_left(wv, shift16), jnp.float32)
                    whi = lax.bitcast_convert_type(
                        lax.bitwise_and(wv, mask_hi), jnp.float32)
                    slo = pl.ds(j * 16, 16)
                    shi = pl.ds(64 + j * 16, 16)
                    gbuf[b, r, slo] = gbuf[b, r, slo] * wlo
                    gbuf[b, r, shi] = gbuf[b, r, shi] * whi

            @pl.when(c + 2 < NCHUNK)
            def _():
                pltpu.async_copy(
                    w_hbm.at[cid, sid, pl.ds((c + 2) * CH, CH)],
                    wbuf.at[ws], wsem)

            @pl.when(c >= 2)
            def _():
                pltpu.make_async_copy(
                    gbuf.at[b2], agg.at[dst_r.at[b2]], ssem).wait()

            if b % 2 == 1:
                @pl.when((c >= 3) & (c + 2 < NCHUNK))
                def _():
                    pltpu.async_copy(
                        dst_hbm.at[cid, sid, pl.ds(c + 1, 2)],
                        dst_r.at[pl.ds((b + 1) % NROT, 2)], dsem)

                @pl.when(c + 4 < NCHUNK)
                def _():
                    pltpu.async_copy(
                        src_hbm.at[cid, sid, pl.ds(c + 3, 2)],
                        src_r.at[pl.ds((b + 3) % NROT, 2)], isem)

            @pl.when(c + 2 < NCHUNK)
            def _():
                if b % 2 == 0:
                    @pl.when(c >= 2)
                    def _():
                        pltpu.make_async_copy(
                            src_hbm.at[cid, sid, pl.ds(c + 2, 2)],
                            src_r.at[pl.ds(b2, 2)], isem).wait()

                pltpu.async_copy(x_hbm.at[src_r.at[b2]], gbuf.at[b2], gsem)

            if b % 2 == 0:
                @pl.when(c >= 4)
                def _():
                    pltpu.make_async_copy(
                        dst_hbm.at[cid, sid, pl.ds(c, 2)],
                        dst_r.at[pl.ds(b, 2)], dsem).wait()

            pltpu.async_copy(gbuf.at[b], agg.at[dst_r.at[b]], ssem, add=True)
        return carry

    lax.fori_loop(0, NCHUNK // NROT, _outer, 0)

    pltpu.make_async_copy(
        gbuf.at[(NCHUNK - 2) % NROT],
        agg.at[dst_r.at[(NCHUNK - 2) % NROT]], ssem).wait()
    pltpu.make_async_copy(
        gbuf.at[(NCHUNK - 1) % NROT],
        agg.at[dst_r.at[(NCHUNK - 1) % NROT]], ssem).wait()
    plsc.subcore_barrier()

    pltpu.sync_copy(agg.at[pl.ds(base, ROWS_PER_TILE)],
                    out_hbm.at[cid, pl.ds(base, ROWS_PER_TILE)])


def _f_body(ni_ref, na_ref, agg_ref, wsc_ref, wl2_ref, o_ref):
    na = na_ref[...]
    nie = ni_ref[...] * na
    aggs = (agg_ref[0] + agg_ref[1]) * na
    o_ref[...] = (jnp.dot(nie, wsc_ref[...], preferred_element_type=jnp.float32)
                  * (C_S / math.sqrt(D))
                  + jnp.dot(aggs, wl2_ref[...], preferred_element_type=jnp.float32)
                  * (C_X / math.sqrt(D)))


def _final(ni, na, agg2, w_sc, w_lin2):
    BN = 2000
    return pl.pallas_call(
        _f_body,
        grid=(N // BN,),
        in_specs=[
            pl.BlockSpec((BN, D), lambda i: (i, 0)),
            pl.BlockSpec((BN, 1), lambda i: (i, 0)),
            pl.BlockSpec((NCORE, BN, D), lambda i: (0, i, 0)),
            pl.BlockSpec((D, D), lambda i: (0, 0)),
            pl.BlockSpec((D, D), lambda i: (0, 0)),
        ],
        out_specs=pl.BlockSpec((BN, D), lambda i: (i, 0)),
        out_shape=jax.ShapeDtypeStruct((N, D), jnp.float32),
    )(ni, na, agg2, w_sc, w_lin2)


def kernel(node_input, node_attr, edge_src, edge_dst, edge_attr,
           edge_length_embedded, w_sc, w_lin1, w_lin2, fc_w1, fc_w2):
    pad = EPAD - E
    src4 = jnp.reshape(
        jnp.concatenate([edge_src, jnp.zeros((pad,), jnp.int32)]),
        (NCORE, NSUB, NCHUNK, CH))
    dst4 = jnp.reshape(
        jnp.concatenate([edge_dst, jnp.zeros((pad,), jnp.int32)]),
        (NCORE, NSUB, NCHUNK, CH))
    ea_p = jnp.concatenate([edge_attr, jnp.zeros((pad, 1), jnp.float32)])
    el_p = jnp.concatenate(
        [edge_length_embedded, jnp.zeros((pad, NB), jnp.float32)])

    w_edges, x = _edge_weights()(el_p, ea_p, fc_w1, fc_w2,
                                 node_input, node_attr, w_lin1)
    w4 = jnp.reshape(w_edges, (NCORE, NSUB, EPC, D // 2))
    agg2 = _sc_scatter(x, src4, dst4, w4)
    return _final(node_input, node_attr, agg2[:, :N], w_sc, w_lin2)
